# Initial kernel scaffold; baseline (speedup 1.0000x reference)
import jax
import jax.numpy as jnp
from jax.experimental import pallas as pl


def kernel(x, edge_index, batch, params):
    raise NotImplementedError("write your pallas kernel here")


# speedup vs baseline: 4.1838x; 4.1838x over previous
import functools

import jax
import jax.numpy as jnp
from jax import lax
from jax.experimental import pallas as pl
from jax.experimental.pallas import tpu as pltpu
from jax.experimental.pallas import tpu_sc as plsc

N = 10000
D = 128
E = 320000
G = 64
EPS = 1e-5

NW = 32
EPW = E // NW
CHUNK = 80
NCHUNK = EPW // CHUNK
ROWS_PER_TILE = N // 16
BLK = 2000
NBLK = N // BLK


def _edge_body(k_hbm, q_hbm, v_hbm, src_hbm, dst_hbm, out_hbm,
               acc_sh, z_v, dst_v, src_v, s_v, v_v, m_v, sem_i, sem_g):
    c = lax.axis_index("c")
    s = lax.axis_index("s")
    wid = c * 16 + s

    def _zrow(r, carry):
        for j in range(8):
            z_v[r, pl.ds(j * 16, 16)] = jnp.zeros((16,), jnp.float32)
        return carry
    lax.fori_loop(0, 125, _zrow, 0)
    for t in range(ROWS_PER_TILE // 125):
        pltpu.sync_copy(z_v, acc_sh.at[pl.ds(s * ROWS_PER_TILE + t * 125, 125)])
    plsc.subcore_barrier()

    base_w = wid * EPW

    def _chunk(i, carry):
        eb = base_w + i * CHUNK
        pltpu.sync_copy(dst_hbm.at[pl.ds(eb, CHUNK)], dst_v)
        pltpu.sync_copy(src_hbm.at[pl.ds(eb, CHUNK)], src_v)
        pltpu.async_copy(k_hbm.at[dst_v], s_v, sem_g).wait()
        pltpu.async_copy(q_hbm.at[src_v], s_v, sem_g, add=True).wait()
        pltpu.async_copy(v_hbm.at[src_v], v_v, sem_i).wait()

        def _row(e, c2):
            for j in range(8):
                sv = s_v[e, pl.ds(j * 16, 16)]
                vv = v_v[e, pl.ds(j * 16, 16)]
                m_v[e, pl.ds(j * 16, 16)] = vv / (1.0 + jnp.exp(-sv))
            return c2
        lax.fori_loop(0, CHUNK, _row, 0)

        pltpu.sync_copy(m_v, acc_sh.at[dst_v], add=True)
        return carry

    lax.fori_loop(0, NCHUNK, _chunk, 0)
    plsc.subcore_barrier()

    pltpu.sync_copy(acc_sh.at[pl.ds(s * ROWS_PER_TILE, ROWS_PER_TILE)],
                    out_hbm.at[c, s])


@functools.cache
def _make_edge_sc():
  return pl.kernel(
    _edge_body,
    out_type=jax.ShapeDtypeStruct((2, 16, ROWS_PER_TILE, D), jnp.float32),
    mesh=plsc.VectorSubcoreMesh(core_axis_name="c", subcore_axis_name="s",
                                num_cores=2, num_subcores=16),
    scratch_types=[
        pltpu.VMEM_SHARED((N, D), jnp.float32),
        pltpu.VMEM((125, D), jnp.float32),
        pltpu.VMEM((CHUNK,), jnp.int32),
        pltpu.VMEM((CHUNK,), jnp.int32),
        pltpu.VMEM((CHUNK, D), jnp.float32),
        pltpu.VMEM((CHUNK, D), jnp.float32),
        pltpu.VMEM((CHUNK, D), jnp.float32),
        pltpu.SemaphoreType.DMA,
        pltpu.SemaphoreType.DMA,
    ],
  )


def _edge_sc(k, q, v, src, dst):
    out = _make_edge_sc()(k, q, v, src, dst)
    return out.reshape(2 * N, D)


def _kqvs_first_body(x_ref, w_ref, b_ref, k_ref, q_ref, v_ref, sk_ref):
    y = jnp.dot(x_ref[...], w_ref[...], preferred_element_type=jnp.float32)
    y = y + b_ref[...]
    k_ref[...] = y[:, 0:D]
    q_ref[...] = y[:, D:2 * D]
    v_ref[...] = y[:, 2 * D:3 * D]
    sk_ref[...] = y[:, 3 * D:4 * D]


def _kqvs_bn_body(x_ref, st_ref, g_ref, bb_ref, w_ref, b_ref,
                  k_ref, q_ref, v_ref, sk_ref):
    mean = st_ref[0:1, :] * (1.0 / N)
    var = st_ref[1:2, :] * (1.0 / N) - mean * mean
    h = (x_ref[...] - mean) * lax.rsqrt(var + EPS) * g_ref[...] + bb_ref[...]
    y = jnp.dot(h, w_ref[...], preferred_element_type=jnp.float32)
    y = y + b_ref[...]
    k_ref[...] = y[:, 0:D]
    q_ref[...] = y[:, D:2 * D]
    v_ref[...] = y[:, 2 * D:3 * D]
    sk_ref[...] = y[:, 3 * D:4 * D]


def _res_stats_body(a0_ref, a1_ref, sk_ref, hp_ref, st_ref):
    i = pl.program_id(0)
    hp = a0_ref[...] + a1_ref[...] + sk_ref[...]
    hp_ref[...] = hp
    s1 = jnp.sum(hp, axis=0, keepdims=True)
    s2 = jnp.sum(hp * hp, axis=0, keepdims=True)
    blk = jnp.concatenate([s1, s2, jnp.zeros((6, D), jnp.float32)], axis=0)

    @pl.when(i == 0)
    def _():
        st_ref[...] = blk

    @pl.when(i > 0)
    def _():
        st_ref[...] = st_ref[...] + blk


def _pool_body(hp_ref, st_ref, g_ref, bb_ref, seg_ref,
               gap_ref, gsp_ref, cnt_ref):
    i = pl.program_id(0)
    mean = st_ref[0:1, :] * (1.0 / N)
    var = st_ref[1:2, :] * (1.0 / N) - mean * mean
    h = (hp_ref[...] - mean) * lax.rsqrt(var + EPS) * g_ref[...] + bb_ref[...]
    seg = seg_ref[:, 0:1]
    segT = seg.reshape(1, BLK)
    gid = lax.broadcasted_iota(jnp.int32, (G, BLK), 0)
    mf = (gid == segT).astype(jnp.float32)
    gsum = jnp.dot(mf, h, preferred_element_type=jnp.float32)
    cnt = jnp.broadcast_to(jnp.sum(mf, axis=1, keepdims=True), (G, D))
    rows = [jnp.max(jnp.where(seg == g, h, -jnp.inf), axis=0)
            for g in range(G)]
    gmax = jnp.stack(rows, axis=0)

    @pl.when(i == 0)
    def _():
        gap_ref[...] = gsum
        gsp_ref[...] = gmax
        cnt_ref[...] = cnt

    @pl.when(i > 0)
    def _():
        gap_ref[...] = gap_ref[...] + gsum
        gsp_ref[...] = jnp.maximum(gsp_ref[...], gmax)
        cnt_ref[...] = cnt_ref[...] + cnt


def _bn_rows(x, g, b):
    m = jnp.mean(x, axis=0, keepdims=True)
    v = jnp.mean(x * x, axis=0, keepdims=True) - m * m
    return (x - m) * lax.rsqrt(v + EPS) * g + b


def _mlp_body(gap_ref, gsp_ref, cnt_ref,
              gapg_ref, gapb_ref, gspg_ref, gspb_ref,
              w0_ref, b0_ref, g0_ref, bb0_ref,
              w1_ref, b1_ref, g1_ref, bb1_ref,
              wl_ref, bl_ref, out_ref):
    gap = gap_ref[...] / jnp.maximum(cnt_ref[...], 1.0)
    gap = _bn_rows(gap, gapg_ref[...], gapb_ref[...])
    gsp = _bn_rows(gsp_ref[...], gspg_ref[...], gspb_ref[...])
    out = jnp.concatenate([gap, gsp], axis=1)
    out = jnp.dot(out, w0_ref[...], preferred_element_type=jnp.float32) + b0_ref[...]
    out = (out - jnp.min(out)) / (jnp.max(out) - jnp.min(out))
    out = jnp.maximum(out, 0.0)
    out = _bn_rows(out, g0_ref[...], bb0_ref[...])
    out = jnp.dot(out, w1_ref[...], preferred_element_type=jnp.float32) + b1_ref[...]
    out = (out - jnp.min(out)) / (jnp.max(out) - jnp.min(out))
    out = jnp.maximum(out, 0.0)
    out = _bn_rows(out, g1_ref[...], bb1_ref[...])
    out_ref[...] = (jnp.dot(out, wl_ref[...], preferred_element_type=jnp.float32)
                    + bl_ref[...])


def _row_spec(r, c=D):
    return pl.BlockSpec((r, c), lambda i: (i, 0))


def _full_spec(shape):
    return pl.BlockSpec(shape, lambda i: tuple(0 for _ in shape))


def _kqvs_first(x, wcat, bcat):
    return pl.pallas_call(
        _kqvs_first_body,
        grid=(NBLK,),
        in_specs=[_row_spec(BLK), _full_spec((D, 4 * D)), _full_spec((1, 4 * D))],
        out_specs=[_row_spec(BLK)] * 4,
        out_shape=[jax.ShapeDtypeStruct((N, D), jnp.float32)] * 4,
    )(x, wcat, bcat)


def _kqvs_bn(hp, st, g, b, wcat, bcat):
    return pl.pallas_call(
        _kqvs_bn_body,
        grid=(NBLK,),
        in_specs=[_row_spec(BLK), _full_spec((8, D)), _full_spec((1, D)),
                  _full_spec((1, D)), _full_spec((D, 4 * D)),
                  _full_spec((1, 4 * D))],
        out_specs=[_row_spec(BLK)] * 4,
        out_shape=[jax.ShapeDtypeStruct((N, D), jnp.float32)] * 4,
    )(hp, st, g, b, wcat, bcat)


def _res_stats(agg2, skip):
    return pl.pallas_call(
        _res_stats_body,
        grid=(NBLK,),
        in_specs=[_row_spec(BLK),
                  pl.BlockSpec((BLK, D), lambda i: (i + NBLK, 0)),
                  _row_spec(BLK)],
        out_specs=[_row_spec(BLK), _full_spec((8, D))],
        out_shape=[jax.ShapeDtypeStruct((N, D), jnp.float32),
                   jax.ShapeDtypeStruct((8, D), jnp.float32)],
    )(agg2, agg2, skip)


def _pool(hp, st, g, b, segb):
    return pl.pallas_call(
        _pool_body,
        grid=(NBLK,),
        in_specs=[_row_spec(BLK), _full_spec((8, D)), _full_spec((1, D)),
                  _full_spec((1, D)), _row_spec(BLK)],
        out_specs=[_full_spec((G, D))] * 3,
        out_shape=[jax.ShapeDtypeStruct((G, D), jnp.float32)] * 3,
    )(hp, st, g, b, segb)


def _mlp(gap, gsp, cnt, p):
    w0 = p['lin0_W']
    w1 = p['lin1_W']
    wl = jnp.pad(p['last_W'], ((0, 0), (0, D - 10)))
    bl = jnp.pad(p['last_b'], (0, D - 10)).reshape(1, D)
    args = [gap, gsp, cnt,
            p['gap_g'].reshape(1, D), p['gap_b'].reshape(1, D),
            p['gsp_g'].reshape(1, D), p['gsp_b'].reshape(1, D),
            w0, p['lin0_b'].reshape(1, D),
            p['hbn0_g'].reshape(1, D), p['hbn0_b'].reshape(1, D),
            w1, p['lin1_b'].reshape(1, 64),
            p['hbn1_g'].reshape(1, 64), p['hbn1_b'].reshape(1, 64),
            wl, bl]
    out = pl.pallas_call(
        _mlp_body,
        grid=(1,),
        in_specs=[_full_spec(a.shape) for a in args],
        out_specs=_full_spec((G, D)),
        out_shape=jax.ShapeDtypeStruct((G, D), jnp.float32),
    )(*args)
    return out[:, :10]


def kernel(x, edge_index, batch, params):
    src = edge_index[0]
    dst = edge_index[1]
    segb = jnp.broadcast_to(batch[:, None], (N, D))

    k = q = v = skip = None
    hp = st = None
    for i in range(2):
        wcat = jnp.concatenate([params[f'conv{i}_Wk'], params[f'conv{i}_Wq'],
                                params[f'conv{i}_Wv'], params[f'conv{i}_Wskip']],
                               axis=1)
        bcat = jnp.concatenate([params[f'conv{i}_bk'], params[f'conv{i}_bq'],
                                params[f'conv{i}_bv'], params[f'conv{i}_bias']]
                               ).reshape(1, 4 * D)
        if i == 0:
            k, q, v, skip = _kqvs_first(x, wcat, bcat)
        else:
            k, q, v, skip = _kqvs_bn(hp, st, params[f'cbn{i-1}_g'].reshape(1, D),
                                     params[f'cbn{i-1}_b'].reshape(1, D),
                                     wcat, bcat)
        agg2 = _edge_sc(k, q, v, src, dst)
        hp, st = _res_stats(agg2, skip)

    gap, gsp, cnt = _pool(hp, st, params['cbn1_g'].reshape(1, D),
                          params['cbn1_b'].reshape(1, D), segb)
    return _mlp(gap, gsp, cnt, params)
